# Initial kernel scaffold; baseline (speedup 1.0000x reference)
import jax
import jax.numpy as jnp
from jax.experimental import pallas as pl


def kernel(x, edge_index, rest_lengths):
    raise NotImplementedError("write your pallas kernel here")


# speedup vs baseline: 448.7476x; 448.7476x over previous
import functools

import jax
import jax.numpy as jnp
from jax import lax
from jax.experimental import pallas as pl
from jax.experimental.pallas import tpu as pltpu
from jax.experimental.pallas import tpu_sc as plsc

_ITERS = 6
_MAX_CORR = 0.2
_B = 4
_N = 10000
_E = 640000

_L = 16
_NPAD = 10240
_SLOTS = 8
_EPT = _E // _SLOTS
_CHUNK = 2000
_NCHUNK = _EPT // _CHUNK
_STEPS = _CHUNK // _L
_SW = _NPAD // _SLOTS


def _tile_body(epk, xin, xout, xx, xy, xz, ax, ay, az, ebuf, rbuf, shx, shacc):
    c = lax.axis_index("c")
    s = lax.axis_index("s")
    bl = s >> 3
    slot = s & 7
    b = c * 2 + bl

    pltpu.sync_copy(xin.at[b * 3 + 0], xx)
    pltpu.sync_copy(xin.at[b * 3 + 1], xy)
    pltpu.sync_copy(xin.at[b * 3 + 2], xz)

    zero16 = jnp.zeros((_L,), jnp.float32)

    @pl.loop(0, _NPAD // _L)
    def _zero(k):
        sl = pl.ds(k * _L, _L)
        ax[sl] = zero16
        ay[sl] = zero16
        az[sl] = zero16

    neg_half = jnp.float32(-1.0 / (2.0 + 1e-9))
    magic = jnp.int32(0x5F3759DF)

    @pl.loop(0, _ITERS)
    def _iteration(_):
        @pl.loop(0, _NCHUNK)
        def _chunk(ch):
            g = slot * _NCHUNK + ch
            pltpu.sync_copy(epk.at[g], ebuf)

            @pl.loop(0, _STEPS)
            def _step(t):
                sl = pl.ds(t * _L, _L)
                i16 = ebuf[0, sl]
                j16 = ebuf[1, sl]
                rl16 = plsc.bitcast(ebuf[2, sl], jnp.float32)
                xi0 = plsc.load_gather(xx, [i16])
                xi1 = plsc.load_gather(xy, [i16])
                xi2 = plsc.load_gather(xz, [i16])
                xj0 = plsc.load_gather(xx, [j16])
                xj1 = plsc.load_gather(xy, [j16])
                xj2 = plsc.load_gather(xz, [j16])
                d0 = xi0 - xj0
                d1 = xi1 - xj1
                d2 = xi2 - xj2
                q = d0 * d0 + d1 * d1 + d2 * d2 + 1e-9
                r = plsc.bitcast(magic - (plsc.bitcast(q, jnp.int32) >> 1),
                                 jnp.float32)
                hq = q * 0.5
                r = r * (1.5 - hq * r * r)
                r = r * (1.5 - hq * r * r)
                r = r * (1.5 - hq * r * r)
                dist = q * r
                tt = ((dist - rl16) * neg_half) / (dist + 1e-9)
                c0 = jnp.clip(d0 * tt, -_MAX_CORR, _MAX_CORR)
                c1 = jnp.clip(d1 * tt, -_MAX_CORR, _MAX_CORR)
                c2 = jnp.clip(d2 * tt, -_MAX_CORR, _MAX_CORR)
                plsc.addupdate_scatter(ax, [j16], c0)
                plsc.addupdate_scatter(ay, [j16], c1)
                plsc.addupdate_scatter(az, [j16], c2)
                plsc.addupdate_scatter(ax, [i16], -c0)
                plsc.addupdate_scatter(ay, [i16], -c1)
                plsc.addupdate_scatter(az, [i16], -c2)

        abase = (bl * _SLOTS + slot) * 3 * _NPAD
        pltpu.sync_copy(ax, shacc.at[pl.ds(abase + 0 * _NPAD, _NPAD)])
        pltpu.sync_copy(ay, shacc.at[pl.ds(abase + 1 * _NPAD, _NPAD)])
        pltpu.sync_copy(az, shacc.at[pl.ds(abase + 2 * _NPAD, _NPAD)])
        plsc.subcore_barrier()

        soff = slot * _SW
        xbase = bl * 3 * _NPAD
        for p, xp in enumerate((xx, xy, xz)):
            @pl.loop(0, _SLOTS)
            def _accum(s2, p=p, xp=xp):
                src = (bl * _SLOTS + s2) * 3 * _NPAD + p * _NPAD + soff
                pltpu.sync_copy(shacc.at[pl.ds(src, _SW)], rbuf)

                @pl.loop(0, _SW // _L)
                def _add(v):
                    sl = pl.ds(soff + v * _L, _L)
                    xp[sl] = xp[sl] + rbuf[pl.ds(v * _L, _L)]

            pltpu.sync_copy(xp.at[pl.ds(soff, _SW)],
                            shx.at[pl.ds(xbase + p * _NPAD + soff, _SW)])
        plsc.subcore_barrier()

        pltpu.sync_copy(shx.at[pl.ds(xbase + 0 * _NPAD, _NPAD)], xx)
        pltpu.sync_copy(shx.at[pl.ds(xbase + 1 * _NPAD, _NPAD)], xy)
        pltpu.sync_copy(shx.at[pl.ds(xbase + 2 * _NPAD, _NPAD)], xz)

        @pl.loop(0, _NPAD // _L)
        def _zero2(k):
            sl = pl.ds(k * _L, _L)
            ax[sl] = zero16
            ay[sl] = zero16
            az[sl] = zero16

        plsc.subcore_barrier()

    @pl.when(slot == 0)
    def _write_out():
        pltpu.sync_copy(xx, xout.at[b * 3 + 0])
        pltpu.sync_copy(xy, xout.at[b * 3 + 1])
        pltpu.sync_copy(xz, xout.at[b * 3 + 2])


@jax.jit
def _run(epk, xin):
    mesh = plsc.VectorSubcoreMesh(core_axis_name="c", subcore_axis_name="s")
    f = pl.kernel(
        _tile_body,
        out_type=jax.ShapeDtypeStruct((_B * 3, _NPAD), jnp.float32),
        mesh=mesh,
        compiler_params=pltpu.CompilerParams(needs_layout_passes=False),
        scratch_types=[
            pltpu.VMEM((_NPAD,), jnp.float32),
            pltpu.VMEM((_NPAD,), jnp.float32),
            pltpu.VMEM((_NPAD,), jnp.float32),
            pltpu.VMEM((_NPAD,), jnp.float32),
            pltpu.VMEM((_NPAD,), jnp.float32),
            pltpu.VMEM((_NPAD,), jnp.float32),
            pltpu.VMEM((3, _CHUNK), jnp.int32),
            pltpu.VMEM((_SW,), jnp.float32),
            pltpu.VMEM_SHARED((2 * 3 * _NPAD,), jnp.float32),
            pltpu.VMEM_SHARED((2 * _SLOTS * 3 * _NPAD,), jnp.float32),
        ],
    )
    return f(epk, xin)


def kernel(x, edge_index, rest_lengths):
    B, N, _ = x.shape
    E = edge_index.shape[1]
    assert (B, N, E) == (_B, _N, _E)

    xt = jnp.transpose(x, (0, 2, 1)).reshape(_B * 3, _N)
    xin = jnp.pad(xt, ((0, 0), (0, _NPAD - _N)))

    ii = edge_index[0].astype(jnp.int32).reshape(_SLOTS * _NCHUNK, _CHUNK)
    jj = edge_index[1].astype(jnp.int32).reshape(_SLOTS * _NCHUNK, _CHUNK)
    rr = lax.bitcast_convert_type(
        rest_lengths.astype(jnp.float32), jnp.int32
    ).reshape(_SLOTS * _NCHUNK, _CHUNK)
    epk = jnp.stack([ii, jj, rr], axis=1)

    out = _run(epk, xin)
    return out.reshape(_B, 3, _NPAD)[:, :, :_N].transpose(0, 2, 1)


# speedup vs baseline: 1354.4980x; 3.0184x over previous
import jax
import jax.numpy as jnp
from jax import lax
from jax.experimental import pallas as pl
from jax.experimental.pallas import tpu as pltpu
from jax.experimental.pallas import tpu_sc as plsc

_ITERS = 6
_MAX_CORR = 0.2
_B = 4
_N = 10000
_E = 640000

_L = 16
_NPAD = 10112
_SLOTS = 8
_EPT = _E // _SLOTS
_CHUNK = 1600
_NCHUNK = _EPT // _CHUNK
_STEPS = _CHUNK // _L
_SW = _NPAD // _SLOTS


def _tile_body(epk, xin, xout, xx, xy, xz, ax, ay, az, eb0, eb1, rb0, rb1,
               shx, shacc, sem0, sem1, semr0, semr1, semw):
    c = lax.axis_index("c")
    s = lax.axis_index("s")
    bl = s >> 3
    slot = s & 7
    b = c * 2 + bl

    pltpu.sync_copy(xin.at[b * 3 + 0], xx)
    pltpu.sync_copy(xin.at[b * 3 + 1], xy)
    pltpu.sync_copy(xin.at[b * 3 + 2], xz)

    zero16 = jnp.zeros((_L,), jnp.float32)

    @pl.loop(0, _NPAD // _L, unroll=8)
    def _zero(k):
        sl = pl.ds(k * _L, _L)
        ax[sl] = zero16
        ay[sl] = zero16
        az[sl] = zero16

    magic = jnp.int32(0x5F3759DF)

    def _edge_pass(ebuf):
        @plsc.parallel_loop(0, _STEPS, unroll=10)
        def _step(t):
            sl = pl.ds(t * _L, _L)
            i16 = ebuf[0, sl]
            j16 = ebuf[1, sl]
            rl16 = plsc.bitcast(ebuf[2, sl], jnp.float32)
            xi0 = plsc.load_gather(xx, [i16])
            xi1 = plsc.load_gather(xy, [i16])
            xi2 = plsc.load_gather(xz, [i16])
            xj0 = plsc.load_gather(xx, [j16])
            xj1 = plsc.load_gather(xy, [j16])
            xj2 = plsc.load_gather(xz, [j16])
            d0 = xi0 - xj0
            d1 = xi1 - xj1
            d2 = xi2 - xj2
            q = d0 * d0 + d1 * d1 + d2 * d2 + 1e-9
            r = plsc.bitcast(magic - (plsc.bitcast(q, jnp.int32) >> 1),
                             jnp.float32)
            hq = q * 0.5
            r = r * (1.5 - hq * r * r)
            r = r * (1.5 - hq * r * r)
            tt = rl16 * r - 0.5
            c0 = jnp.clip(d0 * tt, -_MAX_CORR, _MAX_CORR)
            c1 = jnp.clip(d1 * tt, -_MAX_CORR, _MAX_CORR)
            c2 = jnp.clip(d2 * tt, -_MAX_CORR, _MAX_CORR)
            plsc.addupdate_scatter(ax, [j16], c0)
            plsc.addupdate_scatter(ay, [j16], c1)
            plsc.addupdate_scatter(az, [j16], c2)
            plsc.addupdate_scatter(ax, [i16], -c0)
            plsc.addupdate_scatter(ay, [i16], -c1)
            plsc.addupdate_scatter(az, [i16], -c2)

    gbase = slot * _NCHUNK
    soff = slot * _SW
    xbase = bl * 3 * _NPAD
    abase0 = bl * _SLOTS * 3 * _NPAD
    planes = (xx, xy, xz)
    accs = (ax, ay, az)

    def _stripe_src(p, s2):
        return abase0 + (s2 * 3 + p) * _NPAD + soff

    pltpu.async_copy(epk.at[gbase], eb0, sem0)

    @pl.loop(0, _ITERS)
    def _iteration(it):
        @pl.loop(0, _NCHUNK, step=2)
        def _chunk(ch):
            g = gbase + ch
            pltpu.async_copy(epk.at[g + 1], eb1, sem1)
            pltpu.make_async_copy(epk.at[g], eb0, sem0).wait()
            _edge_pass(eb0)

            @pl.when(ch + 2 < _NCHUNK)
            def _prefetch():
                pltpu.async_copy(epk.at[g + 2], eb0, sem0)

            pltpu.make_async_copy(epk.at[g + 1], eb1, sem1).wait()
            _edge_pass(eb1)

        abase = abase0 + slot * 3 * _NPAD
        dumps = [
            pltpu.async_copy(acc, shacc.at[pl.ds(abase + p * _NPAD, _NPAD)],
                             semw)
            for p, acc in enumerate(accs)
        ]
        for d in dumps:
            d.wait()
        plsc.subcore_barrier()

        units = [(p, s2) for p in range(3) for s2 in range(_SLOTS)]
        bufs = (rb0, rb1)
        sems = (semr0, semr1)
        pltpu.async_copy(shacc.at[pl.ds(_stripe_src(*units[0]), _SW)],
                         rb0, semr0)
        for u, (p, s2) in enumerate(units):
            if u + 1 < len(units):
                pltpu.async_copy(
                    shacc.at[pl.ds(_stripe_src(*units[u + 1]), _SW)],
                    bufs[(u + 1) % 2], sems[(u + 1) % 2])
            pltpu.make_async_copy(
                shacc.at[pl.ds(_stripe_src(p, s2), _SW)],
                bufs[u % 2], sems[u % 2]).wait()
            xp = planes[p]
            buf = bufs[u % 2]

            @pl.loop(0, _SW // _L, unroll=8)
            def _add(v, xp=xp, buf=buf):
                sl = pl.ds(soff + v * _L, _L)
                xp[sl] = xp[sl] + buf[pl.ds(v * _L, _L)]

        writes = [
            pltpu.async_copy(xp.at[pl.ds(soff, _SW)],
                             shx.at[pl.ds(xbase + p * _NPAD + soff, _SW)],
                             semw)
            for p, xp in enumerate(planes)
        ]
        for d in writes:
            d.wait()
        plsc.subcore_barrier()

        reads = [
            pltpu.async_copy(shx.at[pl.ds(xbase + p * _NPAD, _NPAD)], xp,
                             semw)
            for p, xp in enumerate(planes)
        ]

        @pl.when(it + 1 < _ITERS)
        def _prefetch_next():
            pltpu.async_copy(epk.at[gbase], eb0, sem0)

        @pl.loop(0, _NPAD // _L, unroll=8)
        def _zero2(k):
            sl = pl.ds(k * _L, _L)
            ax[sl] = zero16
            ay[sl] = zero16
            az[sl] = zero16

        for d in reads:
            d.wait()
        plsc.subcore_barrier()

    @pl.when(slot == 0)
    def _write_out():
        pltpu.sync_copy(xx, xout.at[b * 3 + 0])
        pltpu.sync_copy(xy, xout.at[b * 3 + 1])
        pltpu.sync_copy(xz, xout.at[b * 3 + 2])


@jax.jit
def _run(epk, xin):
    mesh = plsc.VectorSubcoreMesh(core_axis_name="c", subcore_axis_name="s")
    f = pl.kernel(
        _tile_body,
        out_type=jax.ShapeDtypeStruct((_B * 3, _NPAD), jnp.float32),
        mesh=mesh,
        compiler_params=pltpu.CompilerParams(needs_layout_passes=False),
        scratch_types=[
            pltpu.VMEM((_NPAD,), jnp.float32),
            pltpu.VMEM((_NPAD,), jnp.float32),
            pltpu.VMEM((_NPAD,), jnp.float32),
            pltpu.VMEM((_NPAD,), jnp.float32),
            pltpu.VMEM((_NPAD,), jnp.float32),
            pltpu.VMEM((_NPAD,), jnp.float32),
            pltpu.VMEM((3, _CHUNK), jnp.int32),
            pltpu.VMEM((3, _CHUNK), jnp.int32),
            pltpu.VMEM((_SW,), jnp.float32),
            pltpu.VMEM((_SW,), jnp.float32),
            pltpu.VMEM_SHARED((2 * 3 * _NPAD,), jnp.float32),
            pltpu.VMEM_SHARED((2 * _SLOTS * 3 * _NPAD,), jnp.float32),
            pltpu.SemaphoreType.DMA,
            pltpu.SemaphoreType.DMA,
            pltpu.SemaphoreType.DMA,
            pltpu.SemaphoreType.DMA,
            pltpu.SemaphoreType.DMA,
        ],
    )
    return f(epk, xin)


def kernel(x, edge_index, rest_lengths):
    B, N, _ = x.shape
    E = edge_index.shape[1]
    assert (B, N, E) == (_B, _N, _E)

    xt = jnp.transpose(x, (0, 2, 1)).reshape(_B * 3, _N)
    xin = jnp.pad(xt, ((0, 0), (0, _NPAD - _N)))

    ii = edge_index[0].astype(jnp.int32).reshape(_SLOTS * _NCHUNK, _CHUNK)
    jj = edge_index[1].astype(jnp.int32).reshape(_SLOTS * _NCHUNK, _CHUNK)
    rr = lax.bitcast_convert_type(
        rest_lengths.astype(jnp.float32) * jnp.float32(0.5), jnp.int32
    ).reshape(_SLOTS * _NCHUNK, _CHUNK)
    epk = jnp.stack([ii, jj, rr], axis=1)

    out = _run(epk, xin)
    return out.reshape(_B, 3, _NPAD)[:, :, :_N].transpose(0, 2, 1)
